# baseline (speedup 1.0000x reference)
import functools

import jax
import jax.numpy as jnp
from jax import lax
from jax.experimental import pallas as pl
from jax.experimental.pallas import tpu as pltpu
from jax.experimental.pallas import tpu_sc as plsc

O_H, O_W = 7, 7
MAXBIN = 10

NEG = float("-inf")


def _bin_bounds(lo, delta, i, n):
    s_raw = lo + (i * delta + (n - 1)) // n
    e = lo + ((i + 1) * delta + (n - 1)) // n
    s = jnp.where(s_raw == e, s_raw - 1, s_raw)
    return s, e


def _tc_body(rois_ref, fmap_ref, out_ref, rm_ref):
    g = pl.program_id(0)
    b = g // 64
    r = g % 64
    H, W = 64, 64
    x1 = (rois_ref[b, r, 0] * W).astype(jnp.int32)
    y1 = (rois_ref[b, r, 1] * H).astype(jnp.int32)
    x2 = (rois_ref[b, r, 2] * W).astype(jnp.int32)
    y2 = (rois_ref[b, r, 3] * H).astype(jnp.int32)
    dx = x2 + 1 - x1
    dy = y2 + 1 - y1

    rms = []
    for oy in range(O_H):
        ys, ye = _bin_bounds(y1, dy, oy, O_H)
        h = ye - ys
        s0 = jnp.clip(ys, 0, H - MAXBIN)
        off = ys - s0
        window = fmap_ref[0, pl.ds(s0, MAXBIN), :, :]
        i = lax.broadcasted_iota(jnp.int32, (MAXBIN, 1, 1), 0)
        m = (i >= off) & (i < off + h)
        rms.append(jnp.max(jnp.where(m, window, NEG), axis=0))
    rm_ref[...] = jnp.stack(rms, axis=1)

    cols = []
    for ox in range(O_W):
        xs, xe = _bin_bounds(x1, dx, ox, O_W)
        w = xe - xs
        s0 = jnp.clip(xs, 0, W - MAXBIN)
        off = xs - s0
        window = rm_ref[pl.ds(s0, MAXBIN), :, :]
        i = lax.broadcasted_iota(jnp.int32, (MAXBIN, 1, 1), 0)
        m = (i >= off) & (i < off + w)
        cols.append(jnp.max(jnp.where(m, window, NEG), axis=0))
    out_ref[0, 0] = jnp.stack(cols, axis=1)


def _kernel_tc(fmaps, rois):
    B, H, W, C = fmaps.shape
    R = rois.shape[1]
    return pl.pallas_call(
        _tc_body,
        grid=(B * R,),
        in_specs=[
            pl.BlockSpec(memory_space=pltpu.SMEM),
            pl.BlockSpec((1, H, W, C), lambda g: (g // 64, 0, 0, 0)),
        ],
        out_specs=pl.BlockSpec(
            (1, 1, O_H, O_W, C), lambda g: (g // 64, g % 64, 0, 0, 0)
        ),
        out_shape=jax.ShapeDtypeStruct((B, R, O_H, O_W, C), jnp.float32),
        scratch_shapes=[pltpu.VMEM((W, O_H, C), jnp.float32)],
    )(rois, fmaps)


SC_NC = 2
SC_NS = 16
SC_NW = SC_NC * SC_NS
SC_L = 16


SC_NBUF = 8
SC_CH = 4


def _sc_body(fmap_hbm, rois_hbm, out_hbm, rois_v, idx_v, buf_v, outbuf_v, sem):
    C = fmap_hbm.shape[1]
    CB = C // SC_L
    wid = lax.axis_index("s") * SC_NC + lax.axis_index("c")
    rpw = rois_hbm.shape[0] // SC_NW
    roi0 = wid * rpw
    pltpu.sync_copy(rois_hbm.at[pl.ds(roi0, rpw)], rois_v)
    lanes = lax.iota(jnp.int32, SC_L)
    negs = jnp.full((SC_L,), NEG, jnp.float32)

    def roi_body(ri, carry):
        g = roi0 + ri
        v = rois_v[ri, pl.ds(0, SC_L)]
        vi = (v * 64.0).astype(jnp.int32)
        x1 = vi[0]
        y1 = vi[1]
        x2 = vi[2]
        y2 = vi[3]
        dx = x2 + 1 - x1
        dy = y2 + 1 - y1
        hc = (dy + 6) // 7
        wc = (dx + 6) // 7
        nc = hc * wc
        nk = (nc + SC_CH - 1) // SC_CH
        mrec = 65536 // wc + 1
        bbase = (g // 64) * 4096
        nt = 49 * nk

        def issue(t):
            cell = t // nk
            k = t - cell * nk
            oy = cell // 7
            ox = cell - oy * 7
            ys_r = y1 + (oy * dy + 6) // 7
            ye = y1 + ((oy + 1) * dy + 6) // 7
            ys = ys_r - (ys_r == ye).astype(jnp.int32)
            xs_r = x1 + (ox * dx + 6) // 7
            xe = x1 + ((ox + 1) * dx + 6) // 7
            xs = xs_r - (xs_r == xe).astype(jnp.int32)
            p = jnp.minimum(k * SC_CH + lanes, nc - 1)
            py = (p * mrec) >> 16
            px = p - py * wc
            yy = jnp.minimum(ys + py, ye - 1)
            xx = jnp.minimum(xs + px, xe - 1)
            slot = t % SC_NBUF
            idx_v[slot, pl.ds(0, SC_L)] = bbase + yy * 64 + xx
            pltpu.async_copy(
                fmap_hbm.at[idx_v.at[slot, pl.ds(0, SC_CH)]],
                buf_v.at[slot],
                sem,
            )

        for t0 in range(SC_NBUF - 1):
            issue(t0)

        def step(t, acc):
            pltpu.make_async_copy(
                fmap_hbm.at[pl.ds(0, SC_L)], buf_v.at[t % SC_NBUF], sem
            ).wait()

            @pl.when(t + (SC_NBUF - 1) < nt)
            def _():
                issue(t + (SC_NBUF - 1))

            slot = t % SC_NBUF
            new = []
            for cb in range(CB):
                m = acc[cb]
                for row in range(SC_CH):
                    m = jnp.maximum(
                        m, buf_v[slot, row, pl.ds(cb * SC_L, SC_L)]
                    )
                new.append(m)
            cell = t // nk
            is_last = t - cell * nk == nk - 1

            @pl.when(is_last)
            def _():
                cb0 = cell * C
                for cb in range(CB):
                    outbuf_v[pl.ds(cb0 + cb * SC_L, SC_L)] = new[cb]

            return tuple(jnp.where(is_last, negs, m) for m in new)

        acc0 = tuple(negs for _ in range(CB))
        lax.fori_loop(0, nt, step, acc0)
        pltpu.sync_copy(outbuf_v, out_hbm.at[pl.ds(g * 49 * C, 49 * C)])
        return carry

    lax.fori_loop(0, rpw, roi_body, 0)


def kernel(fmaps, rois):
    B, H, W, C = fmaps.shape
    R = rois.shape[1]
    fmap_flat = fmaps.reshape(B * H * W, C)
    rois_pad = jnp.pad(
        rois.reshape(B * R, 4), ((0, 0), (0, SC_L - 4))
    )
    rpw = B * R // SC_NW
    mesh = plsc.VectorSubcoreMesh(core_axis_name="c", subcore_axis_name="s")
    run = pl.kernel(
        _sc_body,
        out_type=jax.ShapeDtypeStruct((B * R * 49 * C,), jnp.float32),
        mesh=mesh,
        scratch_types=[
            pltpu.VMEM((rpw, SC_L), jnp.float32),
            pltpu.VMEM((SC_NBUF, SC_L), jnp.int32),
            pltpu.VMEM((SC_NBUF, SC_CH, C), jnp.float32),
            pltpu.VMEM((49 * C,), jnp.float32),
            pltpu.SemaphoreType.DMA,
        ],
    )
    out = run(fmap_flat, rois_pad)
    return out.reshape(B, R, 7, 7, C)


# speedup vs baseline: 1.5364x; 1.5364x over previous
import functools

import jax
import jax.numpy as jnp
from jax import lax
from jax.experimental import pallas as pl
from jax.experimental.pallas import tpu as pltpu
from jax.experimental.pallas import tpu_sc as plsc

O_H, O_W = 7, 7
MAXBIN = 10

NEG = float("-inf")


def _bin_bounds(lo, delta, i, n):
    s_raw = lo + (i * delta + (n - 1)) // n
    e = lo + ((i + 1) * delta + (n - 1)) // n
    s = jnp.where(s_raw == e, s_raw - 1, s_raw)
    return s, e


def _tc_body(rois_ref, fmap_ref, out_ref, rm_ref):
    g = pl.program_id(0)
    b = g // 64
    r = g % 64
    H, W = 64, 64
    x1 = (rois_ref[b, r, 0] * W).astype(jnp.int32)
    y1 = (rois_ref[b, r, 1] * H).astype(jnp.int32)
    x2 = (rois_ref[b, r, 2] * W).astype(jnp.int32)
    y2 = (rois_ref[b, r, 3] * H).astype(jnp.int32)
    dx = x2 + 1 - x1
    dy = y2 + 1 - y1

    rms = []
    for oy in range(O_H):
        ys, ye = _bin_bounds(y1, dy, oy, O_H)
        h = ye - ys
        s0 = jnp.clip(ys, 0, H - MAXBIN)
        off = ys - s0
        window = fmap_ref[0, pl.ds(s0, MAXBIN), :, :]
        i = lax.broadcasted_iota(jnp.int32, (MAXBIN, 1, 1), 0)
        m = (i >= off) & (i < off + h)
        rms.append(jnp.max(jnp.where(m, window, NEG), axis=0))
    rm_ref[...] = jnp.stack(rms, axis=1)

    cols = []
    for ox in range(O_W):
        xs, xe = _bin_bounds(x1, dx, ox, O_W)
        w = xe - xs
        s0 = jnp.clip(xs, 0, W - MAXBIN)
        off = xs - s0
        window = rm_ref[pl.ds(s0, MAXBIN), :, :]
        i = lax.broadcasted_iota(jnp.int32, (MAXBIN, 1, 1), 0)
        m = (i >= off) & (i < off + w)
        cols.append(jnp.max(jnp.where(m, window, NEG), axis=0))
    out_ref[0] = jnp.stack(cols, axis=1)


def _kernel_tc(fmaps, rois, n_rois):
    B, H, W, C = fmaps.shape
    return pl.pallas_call(
        _tc_body,
        grid=(n_rois,),
        in_specs=[
            pl.BlockSpec(memory_space=pltpu.SMEM),
            pl.BlockSpec((1, H, W, C), lambda g: (g // 64, 0, 0, 0)),
        ],
        out_specs=pl.BlockSpec(
            (1, O_H, O_W, C), lambda g: (g, 0, 0, 0)
        ),
        out_shape=jax.ShapeDtypeStruct((n_rois, O_H, O_W, C), jnp.float32),
        scratch_shapes=[pltpu.VMEM((W, O_H, C), jnp.float32)],
    )(rois, fmaps)


SC_NC = 2
SC_NS = 16
SC_NW = SC_NC * SC_NS
SC_L = 16


SC_NBUF = 8
SC_CH = 4


def _sc_body_make(rbase, n_sc):
  def _sc_body(fmap_hbm, rois_hbm, out_hbm, rois_v, idx_v, buf_v, outbuf_v, sem):
    C = fmap_hbm.shape[1]
    CB = C // SC_L
    wid = lax.axis_index("s") * SC_NC + lax.axis_index("c")
    rpw = n_sc // SC_NW
    roi0 = rbase + wid * rpw
    pltpu.sync_copy(rois_hbm.at[pl.ds(roi0, rpw)], rois_v)
    lanes = lax.iota(jnp.int32, SC_L)
    negs = jnp.full((SC_L,), NEG, jnp.float32)

    def roi_body(ri, carry):
        g = roi0 + ri
        v = rois_v[ri, pl.ds(0, SC_L)]
        vi = (v * 64.0).astype(jnp.int32)
        x1 = vi[0]
        y1 = vi[1]
        x2 = vi[2]
        y2 = vi[3]
        dx = x2 + 1 - x1
        dy = y2 + 1 - y1
        hc = (dy + 6) // 7
        wc = (dx + 6) // 7
        nc = hc * wc
        nk = (nc + SC_CH - 1) // SC_CH
        mrec = 65536 // wc + 1
        bbase = (g // 64) * 4096
        nt = 49 * nk

        def issue(t):
            cell = t // nk
            k = t - cell * nk
            oy = cell // 7
            ox = cell - oy * 7
            ys_r = y1 + (oy * dy + 6) // 7
            ye = y1 + ((oy + 1) * dy + 6) // 7
            ys = ys_r - (ys_r == ye).astype(jnp.int32)
            xs_r = x1 + (ox * dx + 6) // 7
            xe = x1 + ((ox + 1) * dx + 6) // 7
            xs = xs_r - (xs_r == xe).astype(jnp.int32)
            p = jnp.minimum(k * SC_CH + lanes, nc - 1)
            py = (p * mrec) >> 16
            px = p - py * wc
            yy = jnp.minimum(ys + py, ye - 1)
            xx = jnp.minimum(xs + px, xe - 1)
            slot = t % SC_NBUF
            idx_v[slot, pl.ds(0, SC_L)] = bbase + yy * 64 + xx
            pltpu.async_copy(
                fmap_hbm.at[idx_v.at[slot, pl.ds(0, SC_CH)]],
                buf_v.at[slot],
                sem,
            )

        for t0 in range(SC_NBUF - 1):
            issue(t0)

        def step(t, acc):
            pltpu.make_async_copy(
                fmap_hbm.at[pl.ds(0, SC_L)], buf_v.at[t % SC_NBUF], sem
            ).wait()

            @pl.when(t + (SC_NBUF - 1) < nt)
            def _():
                issue(t + (SC_NBUF - 1))

            slot = t % SC_NBUF
            new = []
            for cb in range(CB):
                m = acc[cb]
                for row in range(SC_CH):
                    m = jnp.maximum(
                        m, buf_v[slot, row, pl.ds(cb * SC_L, SC_L)]
                    )
                new.append(m)
            cell = t // nk
            is_last = t - cell * nk == nk - 1

            @pl.when(is_last)
            def _():
                cb0 = cell * C
                for cb in range(CB):
                    outbuf_v[pl.ds(cb0 + cb * SC_L, SC_L)] = new[cb]

            return tuple(jnp.where(is_last, negs, m) for m in new)

        acc0 = tuple(negs for _ in range(CB))
        lax.fori_loop(0, nt, step, acc0)
        pltpu.sync_copy(outbuf_v, out_hbm.at[pl.ds((g - rbase) * 49 * C, 49 * C)])
        return carry

    lax.fori_loop(0, rpw, roi_body, 0)

  return _sc_body


NTC = 128


def kernel(fmaps, rois):
    B, H, W, C = fmaps.shape
    R = rois.shape[1]
    n_sc = B * R - NTC
    fmap_flat = fmaps.reshape(B * H * W, C)
    rois_pad = jnp.pad(
        rois.reshape(B * R, 4), ((0, 0), (0, SC_L - 4))
    )
    rpw = n_sc // SC_NW
    mesh = plsc.VectorSubcoreMesh(core_axis_name="c", subcore_axis_name="s")
    run = pl.kernel(
        _sc_body_make(NTC, n_sc),
        out_type=jax.ShapeDtypeStruct((n_sc * 49 * C,), jnp.float32),
        mesh=mesh,
        scratch_types=[
            pltpu.VMEM((rpw, SC_L), jnp.float32),
            pltpu.VMEM((SC_NBUF, SC_L), jnp.int32),
            pltpu.VMEM((SC_NBUF, SC_CH, C), jnp.float32),
            pltpu.VMEM((49 * C,), jnp.float32),
            pltpu.SemaphoreType.DMA,
        ],
    )
    sc_out = run(fmap_flat, rois_pad).reshape(n_sc, O_H, O_W, C)
    tc_out = _kernel_tc(fmaps, rois, NTC)
    out = jnp.concatenate([tc_out, sc_out], axis=0)
    return out.reshape(B, R, O_H, O_W, C)


# speedup vs baseline: 1.5394x; 1.0019x over previous
import functools

import jax
import jax.numpy as jnp
from jax import lax
from jax.experimental import pallas as pl
from jax.experimental.pallas import tpu as pltpu
from jax.experimental.pallas import tpu_sc as plsc

O_H, O_W = 7, 7
MAXBIN = 10

NEG = float("-inf")


def _bin_bounds(lo, delta, i, n):
    s_raw = lo + (i * delta + (n - 1)) // n
    e = lo + ((i + 1) * delta + (n - 1)) // n
    s = jnp.where(s_raw == e, s_raw - 1, s_raw)
    return s, e


def _tc_body(rois_ref, fmap_ref, out_ref, rm_ref):
    g = pl.program_id(0)
    b = g // 64
    r = g % 64
    H, W = 64, 64
    x1 = (rois_ref[b, r, 0] * W).astype(jnp.int32)
    y1 = (rois_ref[b, r, 1] * H).astype(jnp.int32)
    x2 = (rois_ref[b, r, 2] * W).astype(jnp.int32)
    y2 = (rois_ref[b, r, 3] * H).astype(jnp.int32)
    dx = x2 + 1 - x1
    dy = y2 + 1 - y1

    rms = []
    for oy in range(O_H):
        ys, ye = _bin_bounds(y1, dy, oy, O_H)
        acc0 = fmap_ref[0, ys, :, :]

        def ybody(j, acc, ys=ys):
            return jnp.maximum(acc, fmap_ref[0, ys + j, :, :])

        rms.append(lax.fori_loop(1, ye - ys, ybody, acc0))
    rm_ref[...] = jnp.stack(rms, axis=1)

    cols = []
    for ox in range(O_W):
        xs, xe = _bin_bounds(x1, dx, ox, O_W)
        acc0 = rm_ref[xs, :, :]

        def xbody(j, acc, xs=xs):
            return jnp.maximum(acc, rm_ref[xs + j, :, :])

        cols.append(lax.fori_loop(1, xe - xs, xbody, acc0))
    out_ref[0] = jnp.stack(cols, axis=1)


def _kernel_tc(fmaps, rois, n_rois):
    B, H, W, C = fmaps.shape
    return pl.pallas_call(
        _tc_body,
        grid=(n_rois,),
        in_specs=[
            pl.BlockSpec(memory_space=pltpu.SMEM),
            pl.BlockSpec((1, H, W, C), lambda g: (g // 64, 0, 0, 0)),
        ],
        out_specs=pl.BlockSpec(
            (1, O_H, O_W, C), lambda g: (g, 0, 0, 0)
        ),
        out_shape=jax.ShapeDtypeStruct((n_rois, O_H, O_W, C), jnp.float32),
        scratch_shapes=[pltpu.VMEM((W, O_H, C), jnp.float32)],
    )(rois, fmaps)


SC_NC = 2
SC_NS = 16
SC_NW = SC_NC * SC_NS
SC_L = 16


SC_NBUF = 8
SC_CH = 4


def _sc_body_make(rbase, n_sc):
  def _sc_body(fmap_hbm, rois_hbm, out_hbm, rois_v, idx_v, buf_v, outbuf_v, sem):
    C = fmap_hbm.shape[1]
    CB = C // SC_L
    wid = lax.axis_index("s") * SC_NC + lax.axis_index("c")
    rpw = n_sc // SC_NW
    roi0 = rbase + wid * rpw
    pltpu.sync_copy(rois_hbm.at[pl.ds(roi0, rpw)], rois_v)
    lanes = lax.iota(jnp.int32, SC_L)
    negs = jnp.full((SC_L,), NEG, jnp.float32)

    def roi_body(ri, carry):
        g = roi0 + ri
        v = rois_v[ri, pl.ds(0, SC_L)]
        vi = (v * 64.0).astype(jnp.int32)
        x1 = vi[0]
        y1 = vi[1]
        x2 = vi[2]
        y2 = vi[3]
        dx = x2 + 1 - x1
        dy = y2 + 1 - y1
        hc = (dy + 6) // 7
        wc = (dx + 6) // 7
        nc = hc * wc
        nk = (nc + SC_CH - 1) // SC_CH
        mrec = 65536 // wc + 1
        bbase = (g // 64) * 4096
        nt = 49 * nk

        def issue(t):
            cell = t // nk
            k = t - cell * nk
            oy = cell // 7
            ox = cell - oy * 7
            ys_r = y1 + (oy * dy + 6) // 7
            ye = y1 + ((oy + 1) * dy + 6) // 7
            ys = ys_r - (ys_r == ye).astype(jnp.int32)
            xs_r = x1 + (ox * dx + 6) // 7
            xe = x1 + ((ox + 1) * dx + 6) // 7
            xs = xs_r - (xs_r == xe).astype(jnp.int32)
            p = jnp.minimum(k * SC_CH + lanes, nc - 1)
            py = (p * mrec) >> 16
            px = p - py * wc
            yy = jnp.minimum(ys + py, ye - 1)
            xx = jnp.minimum(xs + px, xe - 1)
            slot = t % SC_NBUF
            idx_v[slot, pl.ds(0, SC_L)] = bbase + yy * 64 + xx
            pltpu.async_copy(
                fmap_hbm.at[idx_v.at[slot, pl.ds(0, SC_CH)]],
                buf_v.at[slot],
                sem,
            )

        for t0 in range(SC_NBUF - 1):
            issue(t0)

        def step(t, acc):
            pltpu.make_async_copy(
                fmap_hbm.at[pl.ds(0, SC_L)], buf_v.at[t % SC_NBUF], sem
            ).wait()

            @pl.when(t + (SC_NBUF - 1) < nt)
            def _():
                issue(t + (SC_NBUF - 1))

            slot = t % SC_NBUF
            new = []
            for cb in range(CB):
                m = acc[cb]
                for row in range(SC_CH):
                    m = jnp.maximum(
                        m, buf_v[slot, row, pl.ds(cb * SC_L, SC_L)]
                    )
                new.append(m)
            cell = t // nk
            is_last = t - cell * nk == nk - 1

            @pl.when(is_last)
            def _():
                cb0 = cell * C
                for cb in range(CB):
                    outbuf_v[pl.ds(cb0 + cb * SC_L, SC_L)] = new[cb]

            return tuple(jnp.where(is_last, negs, m) for m in new)

        acc0 = tuple(negs for _ in range(CB))
        lax.fori_loop(0, nt, step, acc0)
        pltpu.sync_copy(outbuf_v, out_hbm.at[pl.ds((g - rbase) * 49 * C, 49 * C)])
        return carry

    lax.fori_loop(0, rpw, roi_body, 0)

  return _sc_body


NTC = 128


def kernel(fmaps, rois):
    B, H, W, C = fmaps.shape
    R = rois.shape[1]
    n_sc = B * R - NTC
    fmap_flat = fmaps.reshape(B * H * W, C)
    rois_pad = jnp.pad(
        rois.reshape(B * R, 4), ((0, 0), (0, SC_L - 4))
    )
    rpw = n_sc // SC_NW
    mesh = plsc.VectorSubcoreMesh(core_axis_name="c", subcore_axis_name="s")
    run = pl.kernel(
        _sc_body_make(NTC, n_sc),
        out_type=jax.ShapeDtypeStruct((n_sc * 49 * C,), jnp.float32),
        mesh=mesh,
        scratch_types=[
            pltpu.VMEM((rpw, SC_L), jnp.float32),
            pltpu.VMEM((SC_NBUF, SC_L), jnp.int32),
            pltpu.VMEM((SC_NBUF, SC_CH, C), jnp.float32),
            pltpu.VMEM((49 * C,), jnp.float32),
            pltpu.SemaphoreType.DMA,
        ],
    )
    sc_out = run(fmap_flat, rois_pad).reshape(n_sc, O_H, O_W, C)
    tc_out = _kernel_tc(fmaps, rois, NTC)
    out = jnp.concatenate([tc_out, sc_out], axis=0)
    return out.reshape(B, R, O_H, O_W, C)


# speedup vs baseline: 1.5563x; 1.0109x over previous
import functools

import jax
import jax.numpy as jnp
from jax import lax
from jax.experimental import pallas as pl
from jax.experimental.pallas import tpu as pltpu
from jax.experimental.pallas import tpu_sc as plsc

O_H, O_W = 7, 7
MAXBIN = 10

NEG = float("-inf")


def _bin_bounds(lo, delta, i, n):
    s_raw = lo + (i * delta + (n - 1)) // n
    e = lo + ((i + 1) * delta + (n - 1)) // n
    s = jnp.where(s_raw == e, s_raw - 1, s_raw)
    return s, e


def _tc_body(rois_ref, fmap_ref, out_ref, rm_ref):
    g = pl.program_id(0)
    b = g // 64
    r = g % 64
    H, W = 64, 64
    x1 = (rois_ref[b, r, 0] * W).astype(jnp.int32)
    y1 = (rois_ref[b, r, 1] * H).astype(jnp.int32)
    x2 = (rois_ref[b, r, 2] * W).astype(jnp.int32)
    y2 = (rois_ref[b, r, 3] * H).astype(jnp.int32)
    dx = x2 + 1 - x1
    dy = y2 + 1 - y1

    rms = []
    for oy in range(O_H):
        ys, ye = _bin_bounds(y1, dy, oy, O_H)
        acc0 = fmap_ref[0, ys, :, :]

        def ybody(j, acc, ys=ys):
            return jnp.maximum(acc, fmap_ref[0, ys + j, :, :])

        rms.append(lax.fori_loop(1, ye - ys, ybody, acc0))
    rm_ref[...] = jnp.stack(rms, axis=1)

    cols = []
    for ox in range(O_W):
        xs, xe = _bin_bounds(x1, dx, ox, O_W)
        acc0 = rm_ref[xs, :, :]

        def xbody(j, acc, xs=xs):
            return jnp.maximum(acc, rm_ref[xs + j, :, :])

        cols.append(lax.fori_loop(1, xe - xs, xbody, acc0))
    out_ref[0] = jnp.stack(cols, axis=1)


def _kernel_tc(fmaps, rois, n_rois):
    B, H, W, C = fmaps.shape
    return pl.pallas_call(
        _tc_body,
        grid=(n_rois,),
        in_specs=[
            pl.BlockSpec(memory_space=pltpu.SMEM),
            pl.BlockSpec((1, H, W, C), lambda g: (g // 64, 0, 0, 0)),
        ],
        out_specs=pl.BlockSpec(
            (1, O_H, O_W, C), lambda g: (g, 0, 0, 0)
        ),
        out_shape=jax.ShapeDtypeStruct((n_rois, O_H, O_W, C), jnp.float32),
        scratch_shapes=[pltpu.VMEM((W, O_H, C), jnp.float32)],
    )(rois, fmaps)


SC_NC = 2
SC_NS = 16
SC_NW = SC_NC * SC_NS
SC_L = 16


SC_NBUF = 8
SC_CH = 4


def _sc_body_make(rbase, n_sc):
  def _sc_body(fmap_hbm, rois_hbm, out_hbm, rois_v, idx_v, buf_v, outbuf_v, sem):
    C = fmap_hbm.shape[1]
    CB = C // SC_L
    wid = lax.axis_index("s") * SC_NC + lax.axis_index("c")
    rpw = n_sc // SC_NW
    roi0 = rbase + wid * rpw
    pltpu.sync_copy(rois_hbm, rois_v)
    lanes = lax.iota(jnp.int32, SC_L)
    negs = jnp.full((SC_L,), NEG, jnp.float32)

    def roi_body(ri, carry):
        g = roi0 + ri
        v = rois_v[g, pl.ds(0, SC_L)]
        vi = (v * 64.0).astype(jnp.int32)
        x1 = vi[0]
        y1 = vi[1]
        x2 = vi[2]
        y2 = vi[3]
        dx = x2 + 1 - x1
        dy = y2 + 1 - y1
        hc = (dy + 6) // 7
        wc = (dx + 6) // 7
        nc = hc * wc
        nk = (nc + SC_CH - 1) // SC_CH
        mrec = 65536 // wc + 1
        bbase = (g // 64) * 4096
        nt = 49 * nk

        def issue(t):
            cell = t // nk
            k = t - cell * nk
            oy = cell // 7
            ox = cell - oy * 7
            ys_r = y1 + (oy * dy + 6) // 7
            ye = y1 + ((oy + 1) * dy + 6) // 7
            ys = ys_r - (ys_r == ye).astype(jnp.int32)
            xs_r = x1 + (ox * dx + 6) // 7
            xe = x1 + ((ox + 1) * dx + 6) // 7
            xs = xs_r - (xs_r == xe).astype(jnp.int32)
            p = jnp.minimum(k * SC_CH + lanes, nc - 1)
            py = (p * mrec) >> 16
            px = p - py * wc
            yy = jnp.minimum(ys + py, ye - 1)
            xx = jnp.minimum(xs + px, xe - 1)
            slot = t % SC_NBUF
            idx_v[slot, pl.ds(0, SC_L)] = bbase + yy * 64 + xx
            pltpu.async_copy(
                fmap_hbm.at[idx_v.at[slot, pl.ds(0, SC_CH)]],
                buf_v.at[slot],
                sem,
            )

        for t0 in range(SC_NBUF - 1):
            issue(t0)

        def step(t, acc):
            pltpu.make_async_copy(
                fmap_hbm.at[pl.ds(0, SC_L)], buf_v.at[t % SC_NBUF], sem
            ).wait()

            @pl.when(t + (SC_NBUF - 1) < nt)
            def _():
                issue(t + (SC_NBUF - 1))

            slot = t % SC_NBUF
            new = []
            for cb in range(CB):
                m = acc[cb]
                for row in range(SC_CH):
                    m = jnp.maximum(
                        m, buf_v[slot, row, pl.ds(cb * SC_L, SC_L)]
                    )
                new.append(m)
            cell = t // nk
            is_last = t - cell * nk == nk - 1

            @pl.when(is_last)
            def _():
                cb0 = cell * C
                for cb in range(CB):
                    outbuf_v[pl.ds(cb0 + cb * SC_L, SC_L)] = new[cb]

            return tuple(jnp.where(is_last, negs, m) for m in new)

        acc0 = tuple(negs for _ in range(CB))
        lax.fori_loop(0, nt, step, acc0)
        pltpu.sync_copy(outbuf_v, out_hbm.at[pl.ds((g - rbase) * 49 * C, 49 * C)])
        return carry

    lax.fori_loop(0, rpw, roi_body, 0)

  return _sc_body


NTC = 160


def kernel(fmaps, rois):
    B, H, W, C = fmaps.shape
    R = rois.shape[1]
    n_sc = B * R - NTC
    fmap_flat = fmaps.reshape(B * H * W, C)
    rois_pad = jnp.pad(
        rois.reshape(B * R, 4), ((0, 0), (0, SC_L - 4))
    )
    rpw = n_sc // SC_NW
    mesh = plsc.VectorSubcoreMesh(core_axis_name="c", subcore_axis_name="s")
    run = pl.kernel(
        _sc_body_make(NTC, n_sc),
        out_type=jax.ShapeDtypeStruct((n_sc * 49 * C,), jnp.float32),
        mesh=mesh,
        scratch_types=[
            pltpu.VMEM((B * R, SC_L), jnp.float32),
            pltpu.VMEM((SC_NBUF, SC_L), jnp.int32),
            pltpu.VMEM((SC_NBUF, SC_CH, C), jnp.float32),
            pltpu.VMEM((49 * C,), jnp.float32),
            pltpu.SemaphoreType.DMA,
        ],
    )
    sc_out = run(fmap_flat, rois_pad).reshape(n_sc, O_H, O_W, C)
    tc_out = _kernel_tc(fmaps, rois, NTC)
    out = jnp.concatenate([tc_out, sc_out], axis=0)
    return out.reshape(B, R, O_H, O_W, C)
